# baseline (speedup 1.0000x reference)
import jax
import jax.numpy as jnp
from jax import lax
from jax.experimental import pallas as pl
from jax.experimental.pallas import tpu as pltpu
from jax.experimental.pallas import tpu_sc as plsc

EMB_DIM = 64
NC = 2
NS = 16
NW = NC * NS

BM = 8192
HALF = 512

HW = EMB_DIM // 2


def _rne_bf16(u):
    return ((u + 0x7FFF + ((u >> 16) & 1)) >> 16) << 16


def _pack_body(xT_ref, out_ref):
    x = xT_ref[...]
    a = _rne_bf16(lax.bitcast_convert_type(x[:HW, :], jnp.uint32))
    b = _rne_bf16(lax.bitcast_convert_type(x[HW:, :], jnp.uint32))
    w = a | (b >> 16)
    out_ref[...] = lax.bitcast_convert_type(w, jnp.float32).T


def _repack(embT):
    n = embT.shape[1]
    return pl.pallas_call(
        _pack_body,
        grid=(pl.cdiv(n, BM),),
        in_specs=[pl.BlockSpec((EMB_DIM, BM), lambda i: (0, i))],
        out_specs=pl.BlockSpec((BM, HW), lambda i: (i, 0)),
        out_shape=jax.ShapeDtypeStruct((n, HW), jnp.float32),
    )(embT)


def _gather_body(table_hbm, idx_hbm, out_hbm, idx_v, rows_v, sem):
    wid = lax.axis_index("s") * NC + lax.axis_index("c")
    b_per_w = idx_v.shape[0]
    base = wid * b_per_w
    pltpu.sync_copy(idx_hbm.at[pl.ds(base, b_per_w)], idx_v)

    for h in range(b_per_w // HALF):
        def body(g, carry, h=h):
            o = pl.multiple_of(h * HALF + g * 16, 16)
            d = pl.multiple_of(g * 16, 16)
            v = idx_v[pl.ds(o, 16)]
            for j in range(16):
                row = v[j]
                pltpu.async_copy(
                    table_hbm.at[pl.ds(row, 1)], rows_v.at[pl.ds(d + j, 1)], sem
                )
            return carry

        lax.fori_loop(0, HALF // 16, body, 0)
        pltpu.make_async_copy(table_hbm.at[pl.ds(0, HALF)], rows_v, sem).wait()
        pltpu.sync_copy(rows_v, out_hbm.at[pl.ds(base + h * HALF, HALF)])


def _sc_gather(table, idx):
    n = idx.shape[0]
    b_per_w = n // NW
    mesh = plsc.VectorSubcoreMesh(core_axis_name="c", subcore_axis_name="s")
    k = pl.kernel(
        _gather_body,
        out_type=jax.ShapeDtypeStruct((n, table.shape[1]), jnp.float32),
        mesh=mesh,
        scratch_types=[
            pltpu.VMEM((b_per_w,), jnp.int32),
            pltpu.VMEM((HALF, table.shape[1]), jnp.float32),
            pltpu.SemaphoreType.DMA,
        ],
    )
    return k(table, idx)


def _mlp_body(eu_ref, ev_ref, w1u_ref, w1v_ref, b1_ref, w2_ref, b2_ref, out_ref):
    au = lax.bitcast_convert_type(eu_ref[...], jnp.uint32)
    av = lax.bitcast_convert_type(ev_ref[...], jnp.uint32)
    euh = lax.bitcast_convert_type((au >> 16) << 16, jnp.float32)
    eul = lax.bitcast_convert_type(au << 16, jnp.float32)
    evh = lax.bitcast_convert_type((av >> 16) << 16, jnp.float32)
    evl = lax.bitcast_convert_type(av << 16, jnp.float32)
    h = (
        jnp.dot(euh, w1u_ref[:HW], preferred_element_type=jnp.float32)
        + jnp.dot(eul, w1u_ref[HW:], preferred_element_type=jnp.float32)
        + jnp.dot(evh, w1v_ref[:HW], preferred_element_type=jnp.float32)
        + jnp.dot(evl, w1v_ref[HW:], preferred_element_type=jnp.float32)
        + b1_ref[...]
    )
    h = jnp.maximum(h, 0.0)
    o = jnp.dot(h, w2_ref[...], preferred_element_type=jnp.float32) + b2_ref[0, 0]
    out_ref[...] = jax.nn.sigmoid(o)


def _mlp(g, w1u, w1v, b1r, w2t, b2r, batch, blk):
    nb = batch // blk
    return pl.pallas_call(
        _mlp_body,
        grid=(nb,),
        in_specs=[
            pl.BlockSpec((blk, HW), lambda i: (i, 0)),
            pl.BlockSpec((blk, HW), lambda i, nb=nb: (i + nb, 0)),
            pl.BlockSpec((EMB_DIM, EMB_DIM), lambda i: (0, 0)),
            pl.BlockSpec((EMB_DIM, EMB_DIM), lambda i: (0, 0)),
            pl.BlockSpec((1, EMB_DIM), lambda i: (0, 0)),
            pl.BlockSpec((EMB_DIM, 1), lambda i: (0, 0)),
            pl.BlockSpec((1, 1), lambda i: (0, 0)),
        ],
        out_specs=pl.BlockSpec((blk, 1), lambda i: (i, 0)),
        out_shape=jax.ShapeDtypeStruct((batch, 1), jnp.float32),
    )(g, g, w1u, w1v, b1r, w2t, b2r)


def kernel(u_ids, v_ids, emb, W1, b1, W2, b2):
    batch = u_ids.shape[0]
    idx = jnp.concatenate([u_ids.astype(jnp.int32), v_ids.astype(jnp.int32)])
    table = _repack(emb.T)
    g = _sc_gather(table, idx)
    out = _mlp(
        g, W1[:, :EMB_DIM].T, W1[:, EMB_DIM:].T,
        b1.reshape(1, EMB_DIM), W2.T, b2.reshape(1, 1), batch, 1024,
    )
    return out[:, 0]


# speedup vs baseline: 1.0911x; 1.0911x over previous
import jax
import jax.numpy as jnp
from jax import lax
from jax.experimental import pallas as pl
from jax.experimental.pallas import tpu as pltpu
from jax.experimental.pallas import tpu_sc as plsc

EMB_DIM = 64
NC = 2
NS = 16
NW = NC * NS

BM = 32768
HALF = 512

HW = EMB_DIM // 2


def _rne_bf16(u):
    return ((u + 0x7FFF + ((u >> 16) & 1)) >> 16) << 16


def _pack_body(xT_ref, out_ref):
    x = xT_ref[...]
    a = _rne_bf16(lax.bitcast_convert_type(x[:HW, :], jnp.uint32))
    b = _rne_bf16(lax.bitcast_convert_type(x[HW:, :], jnp.uint32))
    w = a | (b >> 16)
    out_ref[...] = lax.bitcast_convert_type(w, jnp.float32).T


def _repack(embT):
    n = embT.shape[1]
    return pl.pallas_call(
        _pack_body,
        grid=(pl.cdiv(n, BM),),
        in_specs=[pl.BlockSpec((EMB_DIM, BM), lambda i: (0, i))],
        out_specs=pl.BlockSpec((BM, HW), lambda i: (i, 0)),
        out_shape=jax.ShapeDtypeStruct((n, HW), jnp.float32),
    )(embT)


def _gather_body(table_hbm, idx_hbm, out_hbm, idx_v, rows_v, sem):
    wid = lax.axis_index("s") * NC + lax.axis_index("c")
    b_per_w = idx_v.shape[0]
    base = wid * b_per_w
    pltpu.sync_copy(idx_hbm.at[pl.ds(base, b_per_w)], idx_v)

    for h in range(b_per_w // HALF):
        def body(g, carry, h=h):
            o = pl.multiple_of(h * HALF + g * 16, 16)
            d = pl.multiple_of(g * 16, 16)
            v = idx_v[pl.ds(o, 16)]
            for j in range(16):
                row = v[j]
                pltpu.async_copy(
                    table_hbm.at[pl.ds(row, 1)], rows_v.at[pl.ds(d + j, 1)], sem
                )
            return carry

        lax.fori_loop(0, HALF // 16, body, 0)
        pltpu.make_async_copy(table_hbm.at[pl.ds(0, HALF)], rows_v, sem).wait()
        pltpu.sync_copy(rows_v, out_hbm.at[pl.ds(base + h * HALF, HALF)])


def _sc_gather(table, idx):
    n = idx.shape[0]
    b_per_w = n // NW
    mesh = plsc.VectorSubcoreMesh(core_axis_name="c", subcore_axis_name="s")
    k = pl.kernel(
        _gather_body,
        out_type=jax.ShapeDtypeStruct((n, table.shape[1]), jnp.float32),
        mesh=mesh,
        scratch_types=[
            pltpu.VMEM((b_per_w,), jnp.int32),
            pltpu.VMEM((HALF, table.shape[1]), jnp.float32),
            pltpu.SemaphoreType.DMA,
        ],
    )
    return k(table, idx)


def _mlp_body(eu_ref, ev_ref, w1u_ref, w1v_ref, b1_ref, w2_ref, b2_ref, out_ref):
    au = lax.bitcast_convert_type(eu_ref[...], jnp.uint32)
    av = lax.bitcast_convert_type(ev_ref[...], jnp.uint32)
    euh = lax.bitcast_convert_type((au >> 16) << 16, jnp.float32)
    eul = lax.bitcast_convert_type(au << 16, jnp.float32)
    evh = lax.bitcast_convert_type((av >> 16) << 16, jnp.float32)
    evl = lax.bitcast_convert_type(av << 16, jnp.float32)
    h = (
        jnp.dot(euh, w1u_ref[:HW], preferred_element_type=jnp.float32)
        + jnp.dot(eul, w1u_ref[HW:], preferred_element_type=jnp.float32)
        + jnp.dot(evh, w1v_ref[:HW], preferred_element_type=jnp.float32)
        + jnp.dot(evl, w1v_ref[HW:], preferred_element_type=jnp.float32)
        + b1_ref[...]
    )
    h = jnp.maximum(h, 0.0)
    o = jnp.dot(h, w2_ref[...], preferred_element_type=jnp.float32) + b2_ref[0, 0]
    out_ref[...] = jax.nn.sigmoid(o)


def _mlp(g, w1u, w1v, b1r, w2t, b2r, batch, blk):
    nb = batch // blk
    return pl.pallas_call(
        _mlp_body,
        grid=(nb,),
        in_specs=[
            pl.BlockSpec((blk, HW), lambda i: (i, 0)),
            pl.BlockSpec((blk, HW), lambda i, nb=nb: (i + nb, 0)),
            pl.BlockSpec((EMB_DIM, EMB_DIM), lambda i: (0, 0)),
            pl.BlockSpec((EMB_DIM, EMB_DIM), lambda i: (0, 0)),
            pl.BlockSpec((1, EMB_DIM), lambda i: (0, 0)),
            pl.BlockSpec((EMB_DIM, 1), lambda i: (0, 0)),
            pl.BlockSpec((1, 1), lambda i: (0, 0)),
        ],
        out_specs=pl.BlockSpec((blk, 1), lambda i: (i, 0)),
        out_shape=jax.ShapeDtypeStruct((batch, 1), jnp.float32),
    )(g, g, w1u, w1v, b1r, w2t, b2r)


def kernel(u_ids, v_ids, emb, W1, b1, W2, b2):
    batch = u_ids.shape[0]
    idx = jnp.concatenate([u_ids.astype(jnp.int32), v_ids.astype(jnp.int32)])
    table = _repack(emb.T)
    g = _sc_gather(table, idx)
    out = _mlp(
        g, W1[:, :EMB_DIM].T, W1[:, EMB_DIM:].T,
        b1.reshape(1, EMB_DIM), W2.T, b2.reshape(1, 1), batch, 1024,
    )
    return out[:, 0]
